# Initial kernel scaffold; baseline (speedup 1.0000x reference)
import jax
import jax.numpy as jnp
from jax.experimental import pallas as pl


def kernel(memory, chapter_indices):
    raise NotImplementedError("write your pallas kernel here")


# speedup vs baseline: 3.0357x; 3.0357x over previous
import jax
import jax.numpy as jnp
from jax import lax
from jax.experimental import pallas as pl
from jax.experimental.pallas import tpu as pltpu
from jax.experimental.pallas import tpu_sc as plsc

_NUM_TOKENS = 512
_DIM = 1024
_NUM_CHAPTERS = 16
_T = 32
_BATCH = 2048
_K = 2
_NPAIRS = _BATCH * _K
_NC = 2
_NS = 16
_NW = _NC * _NS
_PPW = _NPAIRS // _NW
_WINDOW = 8


def _sc_gather_kernel(mem_hbm, cidx_hbm, out_hbm, aidx_hbm,
                      bank, cidx_v, aidx_v, out_sem, in_sem):
    cid = lax.axis_index("c")
    sid = lax.axis_index("s")
    wid = sid * _NC + cid
    base = wid * _PPW

    pltpu.async_copy(cidx_hbm.at[pl.ds(base, _PPW)], cidx_v, in_sem)

    @pl.when(sid == 0)
    def _():
        pltpu.sync_copy(mem_hbm, bank)

    plsc.subcore_barrier()
    pltpu.make_async_copy(cidx_hbm.at[pl.ds(base, _PPW)], cidx_v,
                          in_sem).wait()

    iota = lax.broadcasted_iota(jnp.int32, (16,), 0)
    descs = []
    for g in range(_PPW // 16):
        cvec = cidx_v[pl.ds(g * 16, 16)]
        for l in range(16):
            p = g * 16 + l
            c = cvec[l]
            row0 = c * _T
            lo = row0 + iota
            aidx_v[p, pl.ds(0, 16)] = lo
            aidx_v[p, pl.ds(16, 16)] = lo + 16
            d = pltpu.async_copy(bank.at[pl.ds(row0, _T)],
                                 out_hbm.at[base + p], out_sem)
            descs.append(d)
            if len(descs) > _WINDOW:
                descs.pop(0).wait()

    pltpu.sync_copy(aidx_v, aidx_hbm.at[pl.ds(base, _PPW)])
    for d in descs:
        d.wait()


def kernel(memory, chapter_indices):
    cidx_flat = chapter_indices.reshape(_NPAIRS)
    mesh = plsc.VectorSubcoreMesh(core_axis_name="c", subcore_axis_name="s")
    gathered, aidx = pl.kernel(
        _sc_gather_kernel,
        out_type=(
            jax.ShapeDtypeStruct((_NPAIRS, _T, _DIM), jnp.float32),
            jax.ShapeDtypeStruct((_NPAIRS, _T), jnp.int32),
        ),
        mesh=mesh,
        scratch_types=[
            pltpu.VMEM_SHARED((_NUM_TOKENS, _DIM), jnp.float32),
            pltpu.VMEM((_PPW,), jnp.int32),
            pltpu.VMEM((_PPW, _T), jnp.int32),
            pltpu.SemaphoreType.DMA,
            pltpu.SemaphoreType.DMA,
        ],
    )(memory, cidx_flat)
    return (gathered.reshape(_BATCH, _K * _T, _DIM),
            aidx.reshape(_BATCH, _K * _T).astype(chapter_indices.dtype))
